# Initial kernel scaffold; baseline (speedup 1.0000x reference)
import jax
import jax.numpy as jnp
from jax.experimental import pallas as pl


def kernel(edges, h_input, W_rel, W_self):
    raise NotImplementedError("write your pallas kernel here")


# speedup vs baseline: 1.8430x; 1.8430x over previous
import functools

import jax
import jax.numpy as jnp
from jax.experimental import pallas as pl
from jax.experimental.pallas import tpu as pltpu

_N = 10000
_R = 16
_D = 128


def _transform_body(h_ref, w_ref, o_ref):
    o_ref[0] = jnp.dot(h_ref[...], w_ref[0], preferred_element_type=jnp.float32)


def _final_body(a_ref, h_ref, ws_ref, o_ref):
    o_ref[...] = jax.nn.relu(
        a_ref[...] + jnp.dot(h_ref[...], ws_ref[...], preferred_element_type=jnp.float32)
    )


_BN = 1000


def _transform(h, W_rel):
    return pl.pallas_call(
        _transform_body,
        grid=(_R, _N // _BN),
        in_specs=[
            pl.BlockSpec((_BN, _D), lambda r, nb: (nb, 0)),
            pl.BlockSpec((1, _D, _D), lambda r, nb: (r, 0, 0)),
        ],
        out_specs=pl.BlockSpec((1, _BN, _D), lambda r, nb: (r, nb, 0)),
        out_shape=jax.ShapeDtypeStruct((_R, _N, _D), jnp.float32),
    )(h, W_rel)


def _final(agg, h, W_self):
    return pl.pallas_call(
        _final_body,
        grid=(_N // _BN,),
        in_specs=[
            pl.BlockSpec((_BN, _D), lambda nb: (nb, 0)),
            pl.BlockSpec((_BN, _D), lambda nb: (nb, 0)),
            pl.BlockSpec((_D, _D), lambda nb: (0, 0)),
        ],
        out_specs=pl.BlockSpec((_BN, _D), lambda nb: (nb, 0)),
        out_shape=jax.ShapeDtypeStruct((_N, _D), jnp.float32),
    )(agg, h, W_self)


def kernel(edges, h_input, W_rel, W_self):
    src = edges[:, 0]
    rel = edges[:, 1] % _R
    dst = edges[:, 2]
    transformed = _transform(h_input, W_rel)
    msg = transformed[rel, src]
    pair = dst * _R + rel
    deg = jax.ops.segment_sum(jnp.ones(src.shape, jnp.float32), pair, num_segments=_N * _R)
    norm = 1.0 / jnp.maximum(deg, 1.0)
    msg = msg * norm[pair][:, None]
    agg = jax.ops.segment_sum(msg, dst, num_segments=_N)
    return _final(agg, h_input, W_self)


# speedup vs baseline: 12.1307x; 6.5821x over previous
import functools

import jax
import jax.numpy as jnp
from jax import lax
from jax.experimental import pallas as pl
from jax.experimental.pallas import tpu as pltpu
from jax.experimental.pallas import tpu_sc as plsc

_N = 10000
_R = 16
_D = 128
_E = 320000
_NR = _N * _R
_NC = 2
_NS = 16
_CHUNK = 2000
_B = 80
_ROWS = _CHUNK // _B
_P1_EDGES = _E // _NS
_P3_EDGES = _E // (_NC * _NS)
_NP = 10240
_NODES_T = _NP // _NS
_DEG_T = _NR // _NS


def _transform_body(h_ref, w_ref, o_ref):
    o_ref[0] = jnp.dot(h_ref[...], w_ref[0], preferred_element_type=jnp.float32)


def _final_body(p_ref, h_ref, ws_ref, o_ref):
    o_ref[...] = jax.nn.relu(
        p_ref[0] + p_ref[1]
        + jnp.dot(h_ref[...], ws_ref[...], preferred_element_type=jnp.float32)
    )


_BN = 1000


def _transform(h, W_rel):
    return pl.pallas_call(
        _transform_body,
        grid=(_R, _N // _BN),
        in_specs=[
            pl.BlockSpec((_BN, _D), lambda r, nb: (nb, 0)),
            pl.BlockSpec((1, _D, _D), lambda r, nb: (r, 0, 0)),
        ],
        out_specs=pl.BlockSpec((1, _BN, _D), lambda r, nb: (r, nb, 0)),
        out_shape=jax.ShapeDtypeStruct((_R, _N, _D), jnp.float32),
    )(h, W_rel)


def _final(partials, h, W_self):
    return pl.pallas_call(
        _final_body,
        grid=(_N // _BN,),
        in_specs=[
            pl.BlockSpec((_NC, _BN, _D), lambda nb: (0, nb, 0)),
            pl.BlockSpec((_BN, _D), lambda nb: (nb, 0)),
            pl.BlockSpec((_D, _D), lambda nb: (0, 0)),
        ],
        out_specs=pl.BlockSpec((_BN, _D), lambda nb: (nb, 0)),
        out_shape=jax.ShapeDtypeStruct((_N, _D), jnp.float32),
    )(partials, h, W_self)


_sc_mesh = plsc.VectorSubcoreMesh(core_axis_name="c", subcore_axis_name="s")


@functools.partial(
    pl.kernel,
    mesh=_sc_mesh,
    out_type=jax.ShapeDtypeStruct((_NC, _NP, _D), jnp.float32),
    scratch_types=[
        pltpu.VMEM_SHARED((_NP, _D), jnp.float32),
        pltpu.VMEM_SHARED((_NR,), jnp.float32),
        pltpu.VMEM((_CHUNK,), jnp.int32),
        pltpu.VMEM((_CHUNK,), jnp.int32),
        pltpu.VMEM((_CHUNK,), jnp.int32),
        pltpu.VMEM((_ROWS, _B), jnp.int32),
        pltpu.VMEM((_ROWS, _B), jnp.int32),
        pltpu.VMEM((_ROWS, _B), jnp.int32),
        pltpu.VMEM((_B,), jnp.float32),
        pltpu.VMEM((_B, _D), jnp.float32),
        pltpu.VMEM((_CHUNK,), jnp.float32),
        pltpu.VMEM((_B,), jnp.float32),
        pltpu.SemaphoreType.DMA,
    ],
)
def _sc_kernel(src_hbm, rel_hbm, dst_hbm, table_hbm, zagg_hbm,
               out_hbm,
               agg_sh, deg_sh, relv, dstv, srcv, pair2, didx2, gidx2,
               norm1, rows, degv, ones, sem):
    c = lax.axis_index("c")
    s = lax.axis_index("s")

    pltpu.sync_copy(zagg_hbm.at[pl.ds(s * _NODES_T, _NODES_T)],
                    agg_sh.at[pl.ds(s * _NODES_T, _NODES_T)])
    def zero_vec(j, carry):
        degv[pl.ds(j * 16, 16)] = jnp.zeros((16,), jnp.float32)
        return carry

    lax.fori_loop(0, _CHUNK // 16, zero_vec, 0)

    def zero_deg(k, carry):
        pltpu.sync_copy(degv, deg_sh.at[pl.ds(s * _DEG_T + k * _CHUNK, _CHUNK)])
        return carry

    lax.fori_loop(0, _DEG_T // _CHUNK, zero_deg, 0)
    for q in range(_B // 16):
        ones[pl.ds(q * 16, 16)] = jnp.full((16,), 1.0, jnp.float32)
    plsc.subcore_barrier()

    def p1_chunk(k, carry):
        e0 = s * _P1_EDGES + k * _CHUNK
        pltpu.sync_copy(rel_hbm.at[pl.ds(e0, _CHUNK)], relv)
        pltpu.sync_copy(dst_hbm.at[pl.ds(e0, _CHUNK)], dstv)

        def p1_row(r, carry2):
            for q in range(_B // 16):
                o = r * _B + q * 16
                pr = dstv[pl.ds(o, 16)] * _R + relv[pl.ds(o, 16)]
                pair2[r, pl.ds(q * 16, 16)] = pr
            return carry2

        lax.fori_loop(0, _ROWS, p1_row, 0)

        def p1_scat(r, carry2):
            pltpu.sync_copy(ones, deg_sh.at[pair2.at[r]], add=True)
            return carry2

        lax.fori_loop(0, _ROWS, p1_scat, 0)
        return carry

    lax.fori_loop(0, _P1_EDGES // _CHUNK, p1_chunk, 0)
    plsc.subcore_barrier()

    base = s * _DEG_T

    def p2_chunk(k, carry):
        pltpu.sync_copy(deg_sh.at[pl.ds(base + k * _CHUNK, _CHUNK)], degv)

        def p2_vec(j, carry2):
            v = degv[pl.ds(j * 16, 16)]
            degv[pl.ds(j * 16, 16)] = 1.0 / jnp.maximum(v, 1.0)
            return carry2

        lax.fori_loop(0, _CHUNK // 16, p2_vec, 0)
        pltpu.sync_copy(degv, deg_sh.at[pl.ds(base + k * _CHUNK, _CHUNK)])
        return carry

    lax.fori_loop(0, _DEG_T // _CHUNK, p2_chunk, 0)
    plsc.subcore_barrier()

    def p3_chunk(k, carry):
        e0 = c * (_E // _NC) + s * _P3_EDGES + k * _CHUNK
        pltpu.sync_copy(src_hbm.at[pl.ds(e0, _CHUNK)], srcv)
        pltpu.sync_copy(rel_hbm.at[pl.ds(e0, _CHUNK)], relv)
        pltpu.sync_copy(dst_hbm.at[pl.ds(e0, _CHUNK)], dstv)

        def p3_idx(r, carry2):
            for q in range(_B // 16):
                o = r * _B + q * 16
                rl = relv[pl.ds(o, 16)]
                sr = srcv[pl.ds(o, 16)]
                dd = dstv[pl.ds(o, 16)]
                gidx2[r, pl.ds(q * 16, 16)] = rl * _N + sr
                pair2[r, pl.ds(q * 16, 16)] = dd * _R + rl
                didx2[r, pl.ds(q * 16, 16)] = dd
            return carry2

        lax.fori_loop(0, _ROWS, p3_idx, 0)

        def p3_row(r, carry2):
            pltpu.sync_copy(deg_sh.at[pair2.at[r]], norm1)
            pltpu.async_copy(table_hbm.at[gidx2.at[r]], rows, sem).wait()

            def scale(g, carry3):
                n16 = norm1[pl.ds(g * 16, 16)]
                for l in range(16):
                    nb = jnp.full((16,), n16[l], jnp.float32)
                    j = g * 16 + l
                    for q in range(_D // 16):
                        v = rows[j, pl.ds(q * 16, 16)]
                        rows[j, pl.ds(q * 16, 16)] = v * nb
                return carry3

            lax.fori_loop(0, _B // 16, scale, 0)
            pltpu.sync_copy(rows, agg_sh.at[didx2.at[r]], add=True)
            return carry2

        lax.fori_loop(0, _ROWS, p3_row, 0)
        return carry

    lax.fori_loop(0, _P3_EDGES // _CHUNK, p3_chunk, 0)
    plsc.subcore_barrier()

    pltpu.sync_copy(agg_sh.at[pl.ds(s * _NODES_T, _NODES_T)],
                    out_hbm.at[c, pl.ds(s * _NODES_T, _NODES_T)])


def kernel(edges, h_input, W_rel, W_self):
    src = edges[:, 0].astype(jnp.int32)
    rel = (edges[:, 1] % _R).astype(jnp.int32)
    dst = edges[:, 2].astype(jnp.int32)
    transformed = _transform(h_input, W_rel).reshape(_R * _N, _D)
    zagg = jnp.zeros((_NP, _D), jnp.float32)
    partials = _sc_kernel(src, rel, dst, transformed, zagg)
    return _final(partials, h_input, W_self)
